# speedup vs baseline: 2.8015x; 2.8015x over previous
import jax
import jax.numpy as jnp
from jax.experimental import pallas as pl

_ROWS = 65536
_BATCH = 256
_BLOCK_ROWS = 1024
_NUM_BLOCKS = _ROWS // _BLOCK_ROWS


def _copy_body(x_ref, o_ref):
    o_ref[...] = x_ref[...]


def kernel(x):
    return pl.pallas_call(
        _copy_body,
        grid=(_NUM_BLOCKS,),
        in_specs=[pl.BlockSpec((_BLOCK_ROWS, _BATCH), lambda b: (b ^ 1, 0))],
        out_specs=pl.BlockSpec((_BLOCK_ROWS, _BATCH), lambda b: (b, 0)),
        out_shape=jax.ShapeDtypeStruct((_ROWS, _BATCH), x.dtype),
    )(x)


# speedup vs baseline: 2.8446x; 1.0154x over previous
import functools

import jax
import jax.numpy as jnp
from jax import lax
from jax.experimental import pallas as pl
from jax.experimental.pallas import tpu as pltpu
from jax.experimental.pallas import tpu_sc as plsc

_ROWS = 65536
_BATCH = 256
_FLIP = 1024
_NW = 32
_PER_W = _ROWS // _NW
_CH = 128
_NBUF = 3
_NCHUNK = _PER_W // _CH


def _sc_body(x_hbm, o_hbm, *rest):
    bufs = rest[:_NBUF]
    lsems = rest[_NBUF:2 * _NBUF]
    ssems = rest[2 * _NBUF:3 * _NBUF]
    wid = lax.axis_index("s") * 2 + lax.axis_index("c")
    base = wid * _PER_W

    def load(i, b):
        src = pl.multiple_of((base + i * _CH) ^ _FLIP, _CH)
        return pltpu.make_async_copy(
            x_hbm.at[pl.ds(src, _CH)], bufs[b], lsems[b])

    def store(i, b):
        dst = pl.multiple_of(base + i * _CH, _CH)
        return pltpu.make_async_copy(
            bufs[b], o_hbm.at[pl.ds(dst, _CH)], ssems[b])

    for b in range(_NBUF):
        load(b, b).start()
    for i in range(_NCHUNK):
        b = i % _NBUF
        load(i, b).wait()
        store(i, b).start()
        nxt = i + _NBUF
        if nxt < _NCHUNK:
            store(i, b).wait()
            load(nxt, b).start()
    for i in range(_NCHUNK - _NBUF, _NCHUNK):
        store(i, i % _NBUF).wait()


@functools.partial(jax.jit, donate_argnums=())
def _sc_swap(x):
    mesh = plsc.VectorSubcoreMesh(core_axis_name="c", subcore_axis_name="s")
    scratch = [pltpu.VMEM((_CH, _BATCH), jnp.float32) for _ in range(_NBUF)]
    scratch += [pltpu.SemaphoreType.DMA for _ in range(2 * _NBUF)]
    return pl.kernel(
        _sc_body,
        mesh=mesh,
        out_type=jax.ShapeDtypeStruct((_ROWS, _BATCH), jnp.float32),
        scratch_types=scratch,
    )(x)


def kernel(x):
    return _sc_swap(x)
